# baseline (speedup 1.0000x reference)
import functools

import jax
from jax import lax
from jax.experimental import pallas as pl
from jax.experimental.pallas import tpu as pltpu
from jax.experimental.pallas import tpu_sc as plsc

_NUM_CORES = 2
_NUM_SUBCORES = 16
_NUM_WORKERS = _NUM_CORES * _NUM_SUBCORES
_CHUNK_ROWS = 16
_NBUF = 3


def _make_sc_copy(seq_len, n_embd, dtype):
    rows_per_w = seq_len // _NUM_WORKERS
    n_chunks = rows_per_w // _CHUNK_ROWS
    mesh = plsc.VectorSubcoreMesh(
        core_axis_name="c", subcore_axis_name="s"
    )

    @functools.partial(
        pl.kernel,
        mesh=mesh,
        out_type=jax.ShapeDtypeStruct((seq_len, n_embd), dtype),
        scratch_types=(
            [pltpu.VMEM_SHARED((_NBUF, _NUM_SUBCORES, _CHUNK_ROWS, n_embd), dtype)]
            + [pltpu.SemaphoreType.DMA] * (2 * _NBUF)
        ),
    )
    def sc_copy(table_hbm, out_hbm, spbuf, *sems):
        rsems = sems[:_NBUF]
        wsems = sems[_NBUF:]
        s = lax.axis_index("s")
        wid = s * _NUM_CORES + lax.axis_index("c")
        base = wid * rows_per_w

        def read(i):
            return pltpu.make_async_copy(
                table_hbm.at[pl.ds(base + i * _CHUNK_ROWS, _CHUNK_ROWS)],
                spbuf.at[i % _NBUF, s],
                rsems[i % _NBUF],
            )

        def write(i):
            return pltpu.make_async_copy(
                spbuf.at[i % _NBUF, s],
                out_hbm.at[pl.ds(base + i * _CHUNK_ROWS, _CHUNK_ROWS)],
                wsems[i % _NBUF],
            )

        def xfer(i, bi, src_is_table):
            if src_is_table:
                return pltpu.make_async_copy(
                    table_hbm.at[pl.ds(base + i * _CHUNK_ROWS, _CHUNK_ROWS)],
                    spbuf.at[bi, s],
                    rsems[bi],
                )
            return pltpu.make_async_copy(
                spbuf.at[bi, s],
                out_hbm.at[pl.ds(base + i * _CHUNK_ROWS, _CHUNK_ROWS)],
                wsems[bi],
            )

        read(0).start()
        read(1).start()
        read(0).wait()
        write(0).start()
        read(2).start()

        n_blocks = (n_chunks - 1) // _NBUF

        def block(k, _):
            for b in range(_NBUF):
                i = 1 + k * _NBUF + b
                bi = (1 + b) % _NBUF
                xfer(i, bi, True).wait()
                xfer(i, bi, False).start()

                @pl.when(i + _NBUF - 1 < n_chunks)
                def _():
                    xfer(i - 1, b % _NBUF, False).wait()
                    xfer(i + _NBUF - 1, b % _NBUF, True).start()
            return _

        lax.fori_loop(0, n_blocks, block, None)

        for i in range(n_chunks - _NBUF, n_chunks):
            write(i).wait()

    return sc_copy


def kernel(x, emb_weight):
    seq_len = x.shape[1]
    n_embd = emb_weight.shape[1]
    return _make_sc_copy(seq_len, n_embd, emb_weight.dtype)(emb_weight)
